# baseline (device time: 108794 ns/iter reference)
import jax
import jax.numpy as jnp
from jax import lax
from jax.experimental import pallas as pl
from jax.experimental.pallas import tpu as pltpu

N_DEV = 4
HQ = 8
DH = 128
SQ = 2048
D_MODEL = 1024
WINDOW = 128
SCALE = 0.08838834764831843
CHUNK = SQ // N_DEV
KWIN = CHUNK + 2 * WINDOW
HALF = D_MODEL // 2
N_HOPS = N_DEV - 1


def kernel(x, Wq, K_ext, V_ext, Wo):
    xb = x[0].astype(jnp.bfloat16)
    wqb = Wq.astype(jnp.bfloat16)
    wob = Wo.astype(jnp.bfloat16)

    def body(x_ref, wq_ref, kext_ref, vext_ref, wo_ref, out_ref,
             q_ref, ctx_ref, kst_ref, vst_ref, comm_ref, sbuf_ref, ag_ref,
             dma_sems, send_sems, recv_sems, ag_send_sems, ag_recv_sems):
        my = lax.axis_index("i")
        left = (my + N_DEV - 1) % N_DEV
        right = (my + 1) % N_DEV

        kcp = pltpu.make_async_copy(
            kext_ref.at[0, :, pl.ds(my * HQ, HQ), :], kst_ref,
            dma_sems.at[0])
        vcp = pltpu.make_async_copy(
            vext_ref.at[0, :, pl.ds(my * HQ, HQ), :], vst_ref,
            dma_sems.at[1])
        kcp.start()
        vcp.start()

        barrier = pltpu.get_barrier_semaphore()
        for nbr in (left, right):
            pl.semaphore_signal(barrier, inc=1, device_id=(nbr,),
                                device_id_type=pl.DeviceIdType.MESH)
        pl.semaphore_wait(barrier, 2)

        q_ref[...] = (jnp.dot(x_ref[...], wq_ref[...],
                              preferred_element_type=jnp.float32)
                      * SCALE).astype(jnp.bfloat16)
        kcp.wait()
        vcp.wait()

        def compute_chunk(c):
            q0 = c * CHUNK
            kw = jnp.minimum(jnp.maximum(q0 - WINDOW, 0), SQ - KWIN)
            ii = q0 + lax.broadcasted_iota(jnp.int32, (CHUNK, KWIN), 0)
            jj = kw + lax.broadcasted_iota(jnp.int32, (CHUNK, KWIN), 1)
            bias = jnp.where(jnp.abs(ii - jj) <= WINDOW,
                             jnp.float32(0), jnp.float32(-1e9))
            for h in range(HQ):
                ks = kst_ref[pl.ds(kw, KWIN), h, :].astype(jnp.bfloat16)
                vs = vst_ref[pl.ds(kw, KWIN), h, :].astype(jnp.bfloat16)
                qs = q_ref[pl.ds(q0, CHUNK), h * DH:(h + 1) * DH]
                s = lax.dot_general(
                    qs, ks, (((1,), (1,)), ((), ())),
                    preferred_element_type=jnp.float32)
                w = jnp.exp(s + bias)
                wsum = jnp.sum(w, axis=1, keepdims=True)
                ctx = jnp.dot(w.astype(jnp.bfloat16), vs,
                              preferred_element_type=jnp.float32) / wsum
                ctx_ref[pl.ds(q0, CHUNK), h * DH:(h + 1) * DH] = (
                    ctx.astype(jnp.bfloat16))
            out_ref[0, pl.ds(q0, CHUNK), :] = jnp.dot(
                ctx_ref[pl.ds(q0, CHUNK), :], wo_ref[...],
                preferred_element_type=jnp.float32)

        def ring_rdma(ring, hop, src):
            dev = right if ring == 0 else left
            return pltpu.make_async_remote_copy(
                src_ref=src,
                dst_ref=comm_ref.at[ring, hop],
                send_sem=send_sems.at[ring, hop],
                recv_sem=recv_sems.at[ring, hop],
                device_id=(dev,),
                device_id_type=pl.DeviceIdType.MESH)

        def stage_and_start(s_):
            cs0 = (my - s_ + N_DEV) % N_DEV
            cs1 = (my + s_) % N_DEV
            sbuf_ref[0] = out_ref[0, pl.ds(cs0 * CHUNK, CHUNK),
                                  0:HALF].astype(jnp.bfloat16)
            sbuf_ref[1] = out_ref[0, pl.ds(cs1 * CHUNK, CHUNK),
                                  HALF:D_MODEL].astype(jnp.bfloat16)
            r0 = ring_rdma(0, s_, sbuf_ref.at[0])
            r1 = ring_rdma(1, s_, sbuf_ref.at[1])
            r0.start()
            r1.start()
            return r0, r1

        def wait_and_add(s_, r0, r1):
            r0.wait()
            r1.wait()
            cr0 = (my - s_ - 1 + N_DEV) % N_DEV
            cr1 = (my + s_ + 1) % N_DEV
            out_ref[0, pl.ds(cr0 * CHUNK, CHUNK), 0:HALF] = (
                out_ref[0, pl.ds(cr0 * CHUNK, CHUNK), 0:HALF]
                + comm_ref[0, s_].astype(jnp.float32))
            out_ref[0, pl.ds(cr1 * CHUNK, CHUNK), HALF:D_MODEL] = (
                out_ref[0, pl.ds(cr1 * CHUNK, CHUNK), HALF:D_MODEL]
                + comm_ref[1, s_].astype(jnp.float32))

        compute_chunk(my)
        h0 = stage_and_start(0)
        compute_chunk((my + 1) % N_DEV)
        compute_chunk((my + N_DEV - 1) % N_DEV)
        wait_and_add(0, *h0)
        h1 = stage_and_start(1)
        compute_chunk((my + 2) % N_DEV)
        wait_and_add(1, *h1)
        h2 = stage_and_start(2)
        wait_and_add(2, *h2)

        own0 = (my + 1) % N_DEV
        own1 = (my + N_DEV - 1) % N_DEV
        sbuf_ref[0] = out_ref[0, pl.ds(own0 * CHUNK, CHUNK),
                              0:HALF].astype(jnp.bfloat16)
        sbuf_ref[1] = out_ref[0, pl.ds(own1 * CHUNK, CHUNK),
                              HALF:D_MODEL].astype(jnp.bfloat16)
        sends = []
        for d in range(1, N_DEV):
            tgt = (my + d) % N_DEV
            rel = N_DEV - 1 - d
            for half in range(2):
                rdma = pltpu.make_async_remote_copy(
                    src_ref=sbuf_ref.at[half],
                    dst_ref=ag_ref.at[rel, half],
                    send_sem=ag_send_sems.at[rel, half],
                    recv_sem=ag_recv_sems.at[rel, half],
                    device_id=(tgt,),
                    device_id_type=pl.DeviceIdType.MESH)
                rdma.start()
                sends.append(rdma)
        for rel in range(N_DEV - 1):
            sender = (my + rel + 1) % N_DEV
            for half in range(2):
                recv = pltpu.make_async_remote_copy(
                    src_ref=sbuf_ref.at[half],
                    dst_ref=ag_ref.at[rel, half],
                    send_sem=ag_send_sems.at[rel, half],
                    recv_sem=ag_recv_sems.at[rel, half],
                    device_id=(sender,),
                    device_id_type=pl.DeviceIdType.MESH)
                recv.wait_recv()
            c0 = (my + rel + 2) % N_DEV
            c1 = (my + rel) % N_DEV
            out_ref[0, pl.ds(c0 * CHUNK, CHUNK), 0:HALF] = (
                ag_ref[rel, 0].astype(jnp.float32))
            out_ref[0, pl.ds(c1 * CHUNK, CHUNK), HALF:D_MODEL] = (
                ag_ref[rel, 1].astype(jnp.float32))
        for rdma in sends:
            rdma.wait_send()

    out_shape = jax.ShapeDtypeStruct((1, SQ, D_MODEL), jnp.float32)
    return pl.pallas_call(
        body,
        out_shape=out_shape,
        in_specs=[
            pl.BlockSpec(memory_space=pltpu.VMEM),
            pl.BlockSpec(memory_space=pltpu.VMEM),
            pl.BlockSpec(memory_space=pl.ANY),
            pl.BlockSpec(memory_space=pl.ANY),
            pl.BlockSpec(memory_space=pltpu.VMEM),
        ],
        out_specs=pl.BlockSpec(memory_space=pltpu.VMEM),
        scratch_shapes=[
            pltpu.VMEM((SQ, HQ * DH), jnp.bfloat16),
            pltpu.VMEM((SQ, HQ * DH), jnp.bfloat16),
            pltpu.VMEM((SQ, HQ, DH), jnp.float32),
            pltpu.VMEM((SQ, HQ, DH), jnp.float32),
            pltpu.VMEM((2, N_HOPS, CHUNK, HALF), jnp.bfloat16),
            pltpu.VMEM((2, CHUNK, HALF), jnp.bfloat16),
            pltpu.VMEM((N_DEV - 1, 2, CHUNK, HALF), jnp.bfloat16),
            pltpu.SemaphoreType.DMA((2,)),
            pltpu.SemaphoreType.DMA((2, N_HOPS)),
            pltpu.SemaphoreType.DMA((2, N_HOPS)),
            pltpu.SemaphoreType.DMA((N_DEV - 1, 2)),
            pltpu.SemaphoreType.DMA((N_DEV - 1, 2)),
        ],
        compiler_params=pltpu.CompilerParams(
            collective_id=0, vmem_limit_bytes=56 * 1024 * 1024),
    )(xb, wqb, K_ext, V_ext, wob)


# device time: 102881 ns/iter; 1.0575x vs baseline; 1.0575x over previous
import jax
import jax.numpy as jnp
from jax import lax
from jax.experimental import pallas as pl
from jax.experimental.pallas import tpu as pltpu

N_DEV = 4
HQ = 8
DH = 128
SQ = 2048
D_MODEL = 1024
WINDOW = 128
SCALE = 0.08838834764831843
CHUNK = SQ // N_DEV
KWIN = CHUNK + 2 * WINDOW
HALF = D_MODEL // 2
N_HOPS = N_DEV - 1


def kernel(x, Wq, K_ext, V_ext, Wo):
    xb = x[0]
    wqb = Wq.astype(jnp.bfloat16)
    wob = Wo.astype(jnp.bfloat16)

    def body(x_ref, wq_ref, kext_ref, vext_ref, wo_ref, out_ref,
             q_ref, ctx_ref, kst_ref, vst_ref, comm_ref, sbuf_ref, ag_ref,
             dma_sems, send_sems, recv_sems, ag_send_sems, ag_recv_sems):
        my = lax.axis_index("i")
        left = (my + N_DEV - 1) % N_DEV
        right = (my + 1) % N_DEV

        kcp = pltpu.make_async_copy(
            kext_ref.at[0, :, pl.ds(my * HQ, HQ), :], kst_ref,
            dma_sems.at[0])
        vcp = pltpu.make_async_copy(
            vext_ref.at[0, :, pl.ds(my * HQ, HQ), :], vst_ref,
            dma_sems.at[1])
        kcp.start()
        vcp.start()

        barrier = pltpu.get_barrier_semaphore()
        for nbr in (left, right):
            pl.semaphore_signal(barrier, inc=1, device_id=(nbr,),
                                device_id_type=pl.DeviceIdType.MESH)
        pl.semaphore_wait(barrier, 2)

        q_ref[...] = (jnp.dot(x_ref[...].astype(jnp.bfloat16), wq_ref[...],
                              preferred_element_type=jnp.float32)
                      * (SCALE * 1.4426950408889634)).astype(jnp.bfloat16)
        kcp.wait()
        vcp.wait()

        def compute_chunk(c):
            q0 = c * CHUNK
            kw = jnp.minimum(jnp.maximum(q0 - WINDOW, 0), SQ - KWIN)
            ii = q0 + lax.broadcasted_iota(jnp.int32, (CHUNK, KWIN), 0)
            jj = kw + lax.broadcasted_iota(jnp.int32, (CHUNK, KWIN), 1)
            bias = jnp.where(jnp.abs(ii - jj) <= WINDOW,
                             jnp.float32(0), jnp.float32(-1e9))
            for h in range(HQ):
                ks = kst_ref[pl.ds(kw, KWIN), h, :].astype(jnp.bfloat16)
                vs = vst_ref[pl.ds(kw, KWIN), h, :].astype(jnp.bfloat16)
                qs = q_ref[pl.ds(q0, CHUNK), h * DH:(h + 1) * DH]
                s = lax.dot_general(
                    qs, ks, (((1,), (1,)), ((), ())),
                    preferred_element_type=jnp.float32)
                w = jnp.exp2(s + bias)
                wsum = jnp.sum(w, axis=1, keepdims=True)
                ctx = jnp.dot(w.astype(jnp.bfloat16), vs,
                              preferred_element_type=jnp.float32) / wsum
                ctx_ref[pl.ds(q0, CHUNK), h * DH:(h + 1) * DH] = (
                    ctx.astype(jnp.bfloat16))
            out_ref[0, pl.ds(q0, CHUNK), :] = jnp.dot(
                ctx_ref[pl.ds(q0, CHUNK), :], wo_ref[...],
                preferred_element_type=jnp.float32)

        def ring_rdma(ring, hop, src):
            dev = right if ring == 0 else left
            return pltpu.make_async_remote_copy(
                src_ref=src,
                dst_ref=comm_ref.at[ring, hop],
                send_sem=send_sems.at[ring, hop],
                recv_sem=recv_sems.at[ring, hop],
                device_id=(dev,),
                device_id_type=pl.DeviceIdType.MESH)

        def stage_and_start(s_):
            cs0 = (my - s_ + N_DEV) % N_DEV
            cs1 = (my + s_) % N_DEV
            sbuf_ref[0] = out_ref[0, pl.ds(cs0 * CHUNK, CHUNK),
                                  0:HALF].astype(jnp.bfloat16)
            sbuf_ref[1] = out_ref[0, pl.ds(cs1 * CHUNK, CHUNK),
                                  HALF:D_MODEL].astype(jnp.bfloat16)
            r0 = ring_rdma(0, s_, sbuf_ref.at[0])
            r1 = ring_rdma(1, s_, sbuf_ref.at[1])
            r0.start()
            r1.start()
            return r0, r1

        def wait_and_add(s_, r0, r1):
            r0.wait()
            r1.wait()
            cr0 = (my - s_ - 1 + N_DEV) % N_DEV
            cr1 = (my + s_ + 1) % N_DEV
            out_ref[0, pl.ds(cr0 * CHUNK, CHUNK), 0:HALF] = (
                out_ref[0, pl.ds(cr0 * CHUNK, CHUNK), 0:HALF]
                + comm_ref[0, s_].astype(jnp.float32))
            out_ref[0, pl.ds(cr1 * CHUNK, CHUNK), HALF:D_MODEL] = (
                out_ref[0, pl.ds(cr1 * CHUNK, CHUNK), HALF:D_MODEL]
                + comm_ref[1, s_].astype(jnp.float32))

        compute_chunk(my)
        h0 = stage_and_start(0)
        compute_chunk((my + 1) % N_DEV)
        compute_chunk((my + N_DEV - 1) % N_DEV)
        wait_and_add(0, *h0)
        h1 = stage_and_start(1)
        compute_chunk((my + 2) % N_DEV)
        wait_and_add(1, *h1)
        h2 = stage_and_start(2)
        wait_and_add(2, *h2)

        own0 = (my + 1) % N_DEV
        own1 = (my + N_DEV - 1) % N_DEV
        for g in range(N_DEV - 1):
            if g == 0:
                sbuf_ref[0] = out_ref[0, pl.ds(own0 * CHUNK, CHUNK),
                                      0:HALF].astype(jnp.bfloat16)
                sbuf_ref[1] = out_ref[0, pl.ds(own1 * CHUNK, CHUNK),
                                      HALF:D_MODEL].astype(jnp.bfloat16)
                src0, src1 = sbuf_ref.at[0], sbuf_ref.at[1]
            else:
                src0 = ag_ref.at[0, g - 1]
                src1 = ag_ref.at[1, g - 1]
            rr = []
            for ring, src in ((0, src0), (1, src1)):
                dev = right if ring == 0 else left
                rdma = pltpu.make_async_remote_copy(
                    src_ref=src,
                    dst_ref=ag_ref.at[ring, g],
                    send_sem=ag_send_sems.at[ring, g],
                    recv_sem=ag_recv_sems.at[ring, g],
                    device_id=(dev,),
                    device_id_type=pl.DeviceIdType.MESH)
                rdma.start()
                rr.append(rdma)
            rr[0].wait()
            rr[1].wait()
            cr0 = (my - g + N_DEV) % N_DEV
            cr1 = (my + g) % N_DEV
            out_ref[0, pl.ds(cr0 * CHUNK, CHUNK), 0:HALF] = (
                ag_ref[0, g].astype(jnp.float32))
            out_ref[0, pl.ds(cr1 * CHUNK, CHUNK), HALF:D_MODEL] = (
                ag_ref[1, g].astype(jnp.float32))

    out_shape = jax.ShapeDtypeStruct((1, SQ, D_MODEL), jnp.float32)
    return pl.pallas_call(
        body,
        out_shape=out_shape,
        in_specs=[
            pl.BlockSpec(memory_space=pltpu.VMEM),
            pl.BlockSpec(memory_space=pltpu.VMEM),
            pl.BlockSpec(memory_space=pl.ANY),
            pl.BlockSpec(memory_space=pl.ANY),
            pl.BlockSpec(memory_space=pltpu.VMEM),
        ],
        out_specs=pl.BlockSpec(memory_space=pltpu.VMEM),
        scratch_shapes=[
            pltpu.VMEM((SQ, HQ * DH), jnp.bfloat16),
            pltpu.VMEM((SQ, HQ * DH), jnp.bfloat16),
            pltpu.VMEM((SQ, HQ, DH), jnp.float32),
            pltpu.VMEM((SQ, HQ, DH), jnp.float32),
            pltpu.VMEM((2, N_HOPS, CHUNK, HALF), jnp.bfloat16),
            pltpu.VMEM((2, CHUNK, HALF), jnp.bfloat16),
            pltpu.VMEM((2, N_HOPS, CHUNK, HALF), jnp.bfloat16),
            pltpu.SemaphoreType.DMA((2,)),
            pltpu.SemaphoreType.DMA((2, N_HOPS)),
            pltpu.SemaphoreType.DMA((2, N_HOPS)),
            pltpu.SemaphoreType.DMA((2, N_HOPS)),
            pltpu.SemaphoreType.DMA((2, N_HOPS)),
        ],
        compiler_params=pltpu.CompilerParams(
            collective_id=0, vmem_limit_bytes=61 * 1024 * 1024),
    )(xb, wqb, K_ext, V_ext, wob)


# device time: 100724 ns/iter; 1.0801x vs baseline; 1.0214x over previous
import jax
import jax.numpy as jnp
from jax import lax
from jax.experimental import pallas as pl
from jax.experimental.pallas import tpu as pltpu

N_DEV = 4
HQ = 8
DH = 128
SQ = 2048
D_MODEL = 1024
WINDOW = 128
SCALE = 0.08838834764831843
CHUNK = SQ // N_DEV
KWIN = CHUNK + 2 * WINDOW
HALF = D_MODEL // 2
N_HOPS = N_DEV - 1


def kernel(x, Wq, K_ext, V_ext, Wo):
    xb = x[0]
    wqb = Wq.astype(jnp.bfloat16)
    wob = Wo.astype(jnp.bfloat16)

    def body(x_ref, wq_ref, kext_ref, vext_ref, wo_ref, out_ref,
             q_ref, ctx_ref, kst_ref, vst_ref, comm_ref, sbuf_ref,
             slota_ref, slotb_ref, dma_sems, send_sems, recv_sems,
             a_send_sems, a_recv_sems, b_send_sems, b_recv_sems):
        my = lax.axis_index("i")
        left = (my + N_DEV - 1) % N_DEV
        right = (my + 1) % N_DEV

        kcp = pltpu.make_async_copy(
            kext_ref.at[0, :, pl.ds(my * HQ, HQ), :], kst_ref,
            dma_sems.at[0])
        vcp = pltpu.make_async_copy(
            vext_ref.at[0, :, pl.ds(my * HQ, HQ), :], vst_ref,
            dma_sems.at[1])
        kcp.start()
        vcp.start()

        barrier = pltpu.get_barrier_semaphore()
        for nbr in (left, right):
            pl.semaphore_signal(barrier, inc=1, device_id=(nbr,),
                                device_id_type=pl.DeviceIdType.MESH)
        pl.semaphore_wait(barrier, 2)

        q_ref[...] = (jnp.dot(x_ref[...].astype(jnp.bfloat16), wq_ref[...],
                              preferred_element_type=jnp.float32)
                      * (SCALE * 1.4426950408889634)).astype(jnp.bfloat16)
        kcp.wait()
        vcp.wait()

        def compute_chunk(c):
            q0 = c * CHUNK
            kw = jnp.minimum(jnp.maximum(q0 - WINDOW, 0), SQ - KWIN)
            ii = q0 + lax.broadcasted_iota(jnp.int32, (CHUNK, KWIN), 0)
            jj = kw + lax.broadcasted_iota(jnp.int32, (CHUNK, KWIN), 1)
            bias = jnp.where(jnp.abs(ii - jj) <= WINDOW,
                             jnp.float32(0), jnp.float32(-1e9))
            for h in range(HQ):
                ks = kst_ref[pl.ds(kw, KWIN), h, :].astype(jnp.bfloat16)
                vs = vst_ref[pl.ds(kw, KWIN), h, :].astype(jnp.bfloat16)
                qs = q_ref[pl.ds(q0, CHUNK), h * DH:(h + 1) * DH]
                s = lax.dot_general(
                    qs, ks, (((1,), (1,)), ((), ())),
                    preferred_element_type=jnp.float32)
                w = jnp.exp2(s + bias)
                wsum = jnp.sum(w, axis=1, keepdims=True)
                ctx = jnp.dot(w.astype(jnp.bfloat16), vs,
                              preferred_element_type=jnp.float32) / wsum
                ctx_ref[pl.ds(q0, CHUNK), h * DH:(h + 1) * DH] = (
                    ctx.astype(jnp.bfloat16))
            out_ref[0, pl.ds(q0, CHUNK), :] = jnp.dot(
                ctx_ref[pl.ds(q0, CHUNK), :], wo_ref[...],
                preferred_element_type=jnp.float32)

        def ring_rdma(ring, hop, src):
            dev = right if ring == 0 else left
            return pltpu.make_async_remote_copy(
                src_ref=src,
                dst_ref=comm_ref.at[ring, hop],
                send_sem=send_sems.at[ring, hop],
                recv_sem=recv_sems.at[ring, hop],
                device_id=(dev,),
                device_id_type=pl.DeviceIdType.MESH)

        def stage_and_start(s_):
            cs0 = (my - s_ + N_DEV) % N_DEV
            cs1 = (my + s_) % N_DEV
            sbuf_ref[0] = out_ref[0, pl.ds(cs0 * CHUNK, CHUNK),
                                  0:HALF].astype(jnp.bfloat16)
            sbuf_ref[1] = out_ref[0, pl.ds(cs1 * CHUNK, CHUNK),
                                  HALF:D_MODEL].astype(jnp.bfloat16)
            r0 = ring_rdma(0, s_, sbuf_ref.at[0])
            r1 = ring_rdma(1, s_, sbuf_ref.at[1])
            r0.start()
            r1.start()
            return r0, r1

        def wait_and_add(s_, r0, r1):
            r0.wait()
            r1.wait()
            cr0 = (my - s_ - 1 + N_DEV) % N_DEV
            cr1 = (my + s_ + 1) % N_DEV
            out_ref[0, pl.ds(cr0 * CHUNK, CHUNK), 0:HALF] = (
                out_ref[0, pl.ds(cr0 * CHUNK, CHUNK), 0:HALF]
                + comm_ref[0, s_].astype(jnp.float32))
            out_ref[0, pl.ds(cr1 * CHUNK, CHUNK), HALF:D_MODEL] = (
                out_ref[0, pl.ds(cr1 * CHUNK, CHUNK), HALF:D_MODEL]
                + comm_ref[1, s_].astype(jnp.float32))

        compute_chunk(my)
        h0 = stage_and_start(0)
        compute_chunk((my + 1) % N_DEV)
        compute_chunk((my + N_DEV - 1) % N_DEV)
        wait_and_add(0, *h0)
        h1 = stage_and_start(1)
        compute_chunk((my + 2) % N_DEV)
        wait_and_add(1, *h1)
        h2 = stage_and_start(2)
        wait_and_add(2, *h2)

        own0 = (my + 1) % N_DEV
        own1 = (my + N_DEV - 1) % N_DEV
        sbuf_ref[0] = out_ref[0, pl.ds(own0 * CHUNK, CHUNK),
                              0:HALF].astype(jnp.bfloat16)
        sbuf_ref[1] = out_ref[0, pl.ds(own1 * CHUNK, CHUNK),
                              HALF:D_MODEL].astype(jnp.bfloat16)

        def a_rdma(direction, half):
            dev = right if direction == 0 else left
            return pltpu.make_async_remote_copy(
                src_ref=sbuf_ref.at[half],
                dst_ref=slota_ref.at[direction, half],
                send_sem=a_send_sems.at[direction, half],
                recv_sem=a_recv_sems.at[direction, half],
                device_id=(dev,),
                device_id_type=pl.DeviceIdType.MESH)

        a_sends = []
        for direction in range(2):
            for half in range(2):
                rdma = a_rdma(direction, half)
                rdma.start()
                a_sends.append(rdma)

        a_rdma(0, 0).wait_recv()
        a_rdma(1, 1).wait_recv()
        b0 = pltpu.make_async_remote_copy(
            src_ref=slota_ref.at[0, 0], dst_ref=slotb_ref.at[0],
            send_sem=b_send_sems.at[0], recv_sem=b_recv_sems.at[0],
            device_id=(right,), device_id_type=pl.DeviceIdType.MESH)
        b1 = pltpu.make_async_remote_copy(
            src_ref=slota_ref.at[1, 1], dst_ref=slotb_ref.at[1],
            send_sem=b_send_sems.at[1], recv_sem=b_recv_sems.at[1],
            device_id=(left,), device_id_type=pl.DeviceIdType.MESH)
        b0.start()
        b1.start()

        out_ref[0, pl.ds(((my) % N_DEV) * CHUNK, CHUNK), 0:HALF] = (
            slota_ref[0, 0].astype(jnp.float32))
        out_ref[0, pl.ds(((my) % N_DEV) * CHUNK, CHUNK), HALF:D_MODEL] = (
            slota_ref[1, 1].astype(jnp.float32))
        a_rdma(0, 1).wait_recv()
        out_ref[0, pl.ds(((my + 2) % N_DEV) * CHUNK, CHUNK),
                HALF:D_MODEL] = slota_ref[0, 1].astype(jnp.float32)
        a_rdma(1, 0).wait_recv()
        out_ref[0, pl.ds(((my + 2) % N_DEV) * CHUNK, CHUNK), 0:HALF] = (
            slota_ref[1, 0].astype(jnp.float32))

        b0.wait()
        b1.wait()
        out_ref[0, pl.ds(((my + N_DEV - 1) % N_DEV) * CHUNK, CHUNK),
                0:HALF] = slotb_ref[0].astype(jnp.float32)
        out_ref[0, pl.ds(((my + 1) % N_DEV) * CHUNK, CHUNK),
                HALF:D_MODEL] = slotb_ref[1].astype(jnp.float32)
        for rdma in a_sends:
            rdma.wait_send()

    out_shape = jax.ShapeDtypeStruct((1, SQ, D_MODEL), jnp.float32)
    return pl.pallas_call(
        body,
        out_shape=out_shape,
        in_specs=[
            pl.BlockSpec(memory_space=pltpu.VMEM),
            pl.BlockSpec(memory_space=pltpu.VMEM),
            pl.BlockSpec(memory_space=pl.ANY),
            pl.BlockSpec(memory_space=pl.ANY),
            pl.BlockSpec(memory_space=pltpu.VMEM),
        ],
        out_specs=pl.BlockSpec(memory_space=pltpu.VMEM),
        scratch_shapes=[
            pltpu.VMEM((SQ, HQ * DH), jnp.bfloat16),
            pltpu.VMEM((SQ, HQ * DH), jnp.bfloat16),
            pltpu.VMEM((SQ, HQ, DH), jnp.float32),
            pltpu.VMEM((SQ, HQ, DH), jnp.float32),
            pltpu.VMEM((2, N_HOPS, CHUNK, HALF), jnp.bfloat16),
            pltpu.VMEM((2, CHUNK, HALF), jnp.bfloat16),
            pltpu.VMEM((2, 2, CHUNK, HALF), jnp.bfloat16),
            pltpu.VMEM((2, CHUNK, HALF), jnp.bfloat16),
            pltpu.SemaphoreType.DMA((2,)),
            pltpu.SemaphoreType.DMA((2, N_HOPS)),
            pltpu.SemaphoreType.DMA((2, N_HOPS)),
            pltpu.SemaphoreType.DMA((2, 2)),
            pltpu.SemaphoreType.DMA((2, 2)),
            pltpu.SemaphoreType.DMA((2,)),
            pltpu.SemaphoreType.DMA((2,)),
        ],
        compiler_params=pltpu.CompilerParams(
            collective_id=0, vmem_limit_bytes=61 * 1024 * 1024),
    )(xb, wqb, K_ext, V_ext, wob)
